# Initial kernel scaffold; baseline (speedup 1.0000x reference)
import jax
import jax.numpy as jnp
from jax.experimental import pallas as pl


def kernel(x, v_in, g_in, b_in, codebook, v_out, g_out, b_out):
    raise NotImplementedError("write your pallas kernel here")


# speedup vs baseline: 2.0544x; 2.0544x over previous
import jax
import jax.numpy as jnp
from jax import lax
from jax.experimental import pallas as pl
from jax.experimental.pallas import tpu as pltpu
from jax.experimental.pallas import tpu_sc as plsc

_B, _C, _S = 8, 768, 1024
_E, _K = 256, 8192
_TAU = 1.0
_TK = 1024
_KT = _K // _TK
_PREC = lax.Precision.HIGHEST


def _phase1_body(x_ref, v_ref, g_ref, bb_ref, cb_ref,
                 logits_ref, xp_ref, idx_ref,
                 xn_ref, sx_ref, m_ref, a_ref):
    k = pl.program_id(1)

    @pl.when(k == 0)
    def _init():
        v = v_ref[...]
        n = jnp.sqrt(jnp.sum(v * v, axis=1, keepdims=True))
        w = g_ref[...] * v / n
        xp = jnp.dot(w.astype(jnp.bfloat16), x_ref[0].astype(jnp.bfloat16),
                     preferred_element_type=jnp.float32) + bb_ref[...]
        xp_ref[0] = xp
        nrm = jnp.maximum(jnp.sqrt(jnp.sum(xp * xp, axis=0, keepdims=True)),
                          1e-12)
        xn = xp / nrm
        xn_ref[...] = xn.astype(jnp.bfloat16)
        xn2 = xn * xn
        sx_ref[...] = lax.dot_general(
            xn2, jnp.ones((_E, 1), jnp.float32),
            (((0,), (0,)), ((), ())), precision=_PREC,
            preferred_element_type=jnp.float32)
        m_ref[...] = jnp.full((_S, 1), -jnp.inf, jnp.float32)
        a_ref[...] = jnp.zeros((_S, 1), jnp.int32)

    cb = cb_ref[...]
    nc = jnp.maximum(jnp.sqrt(jnp.sum(cb * cb, axis=1, keepdims=True)), 1e-12)
    cn = cb / nc
    cn2 = cn * cn
    sc_row = lax.dot_general(
        jnp.ones((1, _E), jnp.float32), cn2,
        (((1,), (1,)), ((), ())), precision=_PREC,
        preferred_element_type=jnp.float32)
    dots = lax.dot_general(
        xn_ref[...], cn.astype(jnp.bfloat16), (((0,), (1,)), ((), ())),
        preferred_element_type=jnp.float32)
    tile = (2.0 * dots - sx_ref[...] - sc_row) * (1.0 / _TAU)
    logits_ref[...] = tile

    t_max = jnp.max(tile, axis=1, keepdims=True)
    iota = lax.broadcasted_iota(jnp.int32, tile.shape, 1)
    t_arg = jnp.min(jnp.where(tile == t_max, iota, _K), axis=1, keepdims=True)
    better = t_max > m_ref[...]
    a_ref[...] = jnp.where(better, t_arg + k * _TK, a_ref[...])
    m_ref[...] = jnp.where(better, t_max, m_ref[...])

    @pl.when(k == _KT - 1)
    def _fin():
        idx_ref[0] = a_ref[...]


def _phase3_body(xq_ref, xp_ref, v_ref, g_ref, bb_ref, out_ref, loss_ref):
    v = v_ref[...]
    n = jnp.sqrt(jnp.sum(v * v, axis=1, keepdims=True))
    w = g_ref[...] * v / n
    xq = xq_ref[...]
    o = lax.dot_general(w.astype(jnp.bfloat16), xq.astype(jnp.bfloat16),
                        (((1,), (1,)), ((), ())),
                        preferred_element_type=jnp.float32) + bb_ref[...]
    out_ref[0] = o
    xp = xp_ref[0]
    xpt = lax.dot_general(xp, jnp.eye(_E, dtype=jnp.float32),
                          (((0,), (0,)), ((), ())),
                          precision=_PREC,
                          preferred_element_type=jnp.float32)
    d = xpt - xq
    l = jnp.sum(d * d) * (1.0 / float(_E * _S))
    loss_ref[...] = jnp.full((1, 1, 128), l, jnp.float32)


def _phase1(x, v_in, g_in, b_in2, codebook):
    return pl.pallas_call(
        _phase1_body,
        grid=(_B, _KT),
        in_specs=[
            pl.BlockSpec((1, _C, _S), lambda b, k: (b, 0, 0)),
            pl.BlockSpec((_E, _C), lambda b, k: (0, 0)),
            pl.BlockSpec((_E, 1), lambda b, k: (0, 0)),
            pl.BlockSpec((_E, 1), lambda b, k: (0, 0)),
            pl.BlockSpec((_TK, _E), lambda b, k: (k, 0)),
        ],
        out_specs=[
            pl.BlockSpec((_S, _TK), lambda b, k: (b, k)),
            pl.BlockSpec((1, _E, _S), lambda b, k: (b, 0, 0)),
            pl.BlockSpec((1, _S, 1), lambda b, k: (b, 0, 0)),
        ],
        out_shape=[
            jax.ShapeDtypeStruct((_B * _S, _K), jnp.float32),
            jax.ShapeDtypeStruct((_B, _E, _S), jnp.float32),
            jax.ShapeDtypeStruct((_B, _S, 1), jnp.int32),
        ],
        scratch_shapes=[
            pltpu.VMEM((_E, _S), jnp.bfloat16),
            pltpu.VMEM((_S, 1), jnp.float32),
            pltpu.VMEM((_S, 1), jnp.float32),
            pltpu.VMEM((_S, 1), jnp.int32),
        ],
    )(x, v_in, g_in, b_in2, codebook)


def _sc_gather(codebook, idx_row):
    mesh = plsc.VectorSubcoreMesh(core_axis_name="core",
                                  subcore_axis_name="subcore")
    n = idx_row.shape[1]
    window = 128

    @pl.kernel(out_type=jax.ShapeDtypeStruct((n, _E), jnp.float32), mesh=mesh)
    def gk(cb_hbm, i_hbm, o_hbm):
        def body(i_vmem, o_vmem):
            pltpu.sync_copy(cb_hbm.at[i_vmem.at[0]], o_vmem)

        pltpu.emit_pipeline(
            body,
            grid=(n // window,),
            in_specs=[pl.BlockSpec((1, window), lambda i: (0, i))],
            out_specs=[pl.BlockSpec((window, _E), lambda i: (i, 0))],
            core_axis_name=("core", "subcore"),
            dimension_semantics=(pltpu.PARALLEL,),
        )(i_hbm, o_hbm)

    return gk(codebook, idx_row)


def _phase3(xq, xproj, v_out, g_out, b_out2):
    return pl.pallas_call(
        _phase3_body,
        grid=(_B,),
        in_specs=[
            pl.BlockSpec((_S, _E), lambda b: (b, 0)),
            pl.BlockSpec((1, _E, _S), lambda b: (b, 0, 0)),
            pl.BlockSpec((_C, _E), lambda b: (0, 0)),
            pl.BlockSpec((_C, 1), lambda b: (0, 0)),
            pl.BlockSpec((_C, 1), lambda b: (0, 0)),
        ],
        out_specs=[
            pl.BlockSpec((1, _C, _S), lambda b: (b, 0, 0)),
            pl.BlockSpec((1, 1, 128), lambda b: (b, 0, 0)),
        ],
        out_shape=[
            jax.ShapeDtypeStruct((_B, _C, _S), jnp.float32),
            jax.ShapeDtypeStruct((_B, 1, 128), jnp.float32),
        ],
    )(xq, xproj, v_out, g_out, b_out2)


def kernel(x, v_in, g_in, b_in, codebook, v_out, g_out, b_out):
    b_in2 = b_in.reshape(_E, 1)
    b_out2 = b_out.reshape(_C, 1)
    logits, xproj, idx3 = _phase1(x, v_in, g_in, b_in2, codebook)
    xq = _sc_gather(codebook, idx3.reshape(1, _B * _S))
    out, loss3 = _phase3(xq, xproj, v_out, g_out, b_out2)
    x_idxs = idx3.reshape(_B, _S)
    loss = loss3[:, 0, 0]
    return out, logits, x_idxs, loss, loss


# speedup vs baseline: 2.3096x; 1.1242x over previous
import jax
import jax.numpy as jnp
from jax import lax
from jax.experimental import pallas as pl
from jax.experimental.pallas import tpu as pltpu
from jax.experimental.pallas import tpu_sc as plsc

_B, _C, _S = 8, 768, 1024
_E, _K = 256, 8192
_TAU = 1.0
_TK = 1024
_KT = _K // _TK
_NT = (_B * _S) // _S


def _phase1_body(xnb_ref, sx_ref, cnb_ref, sc_ref,
                 logits_ref, idx_ref, m_ref, a_ref):
    k = pl.program_id(1)

    @pl.when(k == 0)
    def _init():
        m_ref[...] = jnp.full((_S, 1), -jnp.inf, jnp.float32)
        a_ref[...] = jnp.zeros((_S, 1), jnp.int32)

    dots = lax.dot_general(
        xnb_ref[...], cnb_ref[...], (((1,), (1,)), ((), ())),
        preferred_element_type=jnp.float32)
    dist = (sx_ref[0] - 2.0 * dots) + sc_ref[0]
    tile = -dist * (1.0 / _TAU)
    logits_ref[...] = tile

    t_max = jnp.max(tile, axis=1, keepdims=True)
    iota = lax.broadcasted_iota(jnp.int32, tile.shape, 1)
    t_arg = jnp.min(jnp.where(tile == t_max, iota, _K), axis=1, keepdims=True)
    better = t_max > m_ref[...]
    a_ref[...] = jnp.where(better, t_arg + k * _TK, a_ref[...])
    m_ref[...] = jnp.where(better, t_max, m_ref[...])

    @pl.when(k == _KT - 1)
    def _fin():
        idx_ref[0] = a_ref[...]


def _phase1(xnb, sxn3, cnb, scn3):
    return pl.pallas_call(
        _phase1_body,
        grid=(_NT, _KT),
        in_specs=[
            pl.BlockSpec((_S, _E), lambda t, k: (t, 0)),
            pl.BlockSpec((1, _S, 1), lambda t, k: (t, 0, 0)),
            pl.BlockSpec((_TK, _E), lambda t, k: (k, 0)),
            pl.BlockSpec((1, 1, _TK), lambda t, k: (k, 0, 0)),
        ],
        out_specs=[
            pl.BlockSpec((_S, _TK), lambda t, k: (t, k)),
            pl.BlockSpec((1, _S, 1), lambda t, k: (t, 0, 0)),
        ],
        out_shape=[
            jax.ShapeDtypeStruct((_B * _S, _K), jnp.float32),
            jax.ShapeDtypeStruct((_NT, _S, 1), jnp.int32),
        ],
        scratch_shapes=[
            pltpu.VMEM((_S, 1), jnp.float32),
            pltpu.VMEM((_S, 1), jnp.int32),
        ],
    )(xnb, sxn3, cnb, scn3)


def _sc_gather(codebook, idx_row):
    mesh = plsc.VectorSubcoreMesh(core_axis_name="core",
                                  subcore_axis_name="subcore")
    n = idx_row.shape[1]
    window = 128

    @pl.kernel(out_type=jax.ShapeDtypeStruct((n, _E), jnp.float32), mesh=mesh)
    def gk(cb_hbm, i_hbm, o_hbm):
        def body(i_vmem, o_vmem):
            pltpu.sync_copy(cb_hbm.at[i_vmem.at[0]], o_vmem)

        pltpu.emit_pipeline(
            body,
            grid=(n // window,),
            in_specs=[pl.BlockSpec((1, window), lambda i: (0, i))],
            out_specs=[pl.BlockSpec((window, _E), lambda i: (i, 0))],
            core_axis_name=("core", "subcore"),
            dimension_semantics=(pltpu.PARALLEL,),
        )(i_hbm, o_hbm)

    return gk(codebook, idx_row)


def _phase3_body(xq_ref, fl_ref, w_ref, bb_ref, out_ref, loss_ref):
    f = fl_ref[...]
    q = xq_ref[...]
    st = f + (q - f)
    o = lax.dot_general(w_ref[...].astype(jnp.bfloat16), st.astype(jnp.bfloat16),
                        (((1,), (1,)), ((), ())),
                        preferred_element_type=jnp.float32) + bb_ref[...]
    out_ref[0] = o
    d = f - q
    l = jnp.sum(d * d) * (1.0 / float(_E * _S))
    loss_ref[...] = jnp.full((1, 1, 128), l, jnp.float32)


def _phase3(xq, flat, w_out, b_out2):
    return pl.pallas_call(
        _phase3_body,
        grid=(_B,),
        in_specs=[
            pl.BlockSpec((_S, _E), lambda b: (b, 0)),
            pl.BlockSpec((_S, _E), lambda b: (b, 0)),
            pl.BlockSpec((_C, _E), lambda b: (0, 0)),
            pl.BlockSpec((_C, 1), lambda b: (0, 0)),
        ],
        out_specs=[
            pl.BlockSpec((1, _C, _S), lambda b: (b, 0, 0)),
            pl.BlockSpec((1, 1, 128), lambda b: (b, 0, 0)),
        ],
        out_shape=[
            jax.ShapeDtypeStruct((_B, _C, _S), jnp.float32),
            jax.ShapeDtypeStruct((_B, 1, 128), jnp.float32),
        ],
    )(xq, flat, w_out, b_out2)


def kernel(x, v_in, g_in, b_in, codebook, v_out, g_out, b_out):
    n_in = jnp.sqrt(jnp.sum(v_in * v_in, axis=1, keepdims=True))
    w_in = g_in * v_in / n_in
    x_proj = jnp.einsum('oc,bcs->bos', w_in, x) + b_in[None, :, None]
    flat = jnp.transpose(x_proj, (0, 2, 1)).reshape(_B * _S, _E)
    xn = flat / jnp.maximum(jnp.linalg.norm(flat, axis=1, keepdims=True), 1e-12)
    cn = codebook / jnp.maximum(jnp.linalg.norm(codebook, axis=1, keepdims=True), 1e-12)
    sxn3 = jnp.sum(xn * xn, axis=1, keepdims=True).reshape(_NT, _S, 1)
    scn3 = jnp.sum(cn * cn, axis=1, keepdims=True).reshape(_KT, 1, _TK)
    n_out = jnp.sqrt(jnp.sum(v_out * v_out, axis=1, keepdims=True))
    w_out = g_out * v_out / n_out
    b_out2 = b_out.reshape(_C, 1)

    logits, idx3 = _phase1(xn.astype(jnp.bfloat16), sxn3,
                           cn.astype(jnp.bfloat16), scn3)
    xq = _sc_gather(codebook, idx3.reshape(1, _B * _S))
    out, loss3 = _phase3(xq, flat, w_out, b_out2)
    x_idxs = idx3.reshape(_B, _S)
    loss = loss3[:, 0, 0]
    return out, logits, x_idxs, loss, loss


# speedup vs baseline: 2.4199x; 1.0477x over previous
import jax
import jax.numpy as jnp
from jax import lax
from jax.experimental import pallas as pl
from jax.experimental.pallas import tpu as pltpu
from jax.experimental.pallas import tpu_sc as plsc

_B, _C, _S = 8, 768, 1024
_E, _K = 256, 8192
_TAU = 1.0
_TK = 1024
_KT = _K // _TK
_NT = (_B * _S) // _S


def _phase1_body(xnb_ref, sx_ref, cnb_ref, sc_ref, io_ref,
                 logits_ref, idx_ref, m_ref, a_ref):
    k = pl.program_id(1)

    @pl.when(k == 0)
    def _init():
        m_ref[...] = jnp.full((_S, 1), -jnp.inf, jnp.float32)
        a_ref[...] = jnp.zeros((_S, 1), jnp.float32)

    dots = lax.dot_general(
        xnb_ref[...], cnb_ref[...], (((1,), (1,)), ((), ())),
        preferred_element_type=jnp.float32)
    tile = (2.0 * dots - sx_ref[0]) - sc_ref[0]
    logits_ref[...] = tile

    t_max = jnp.max(tile, axis=1, keepdims=True)
    t_arg = jnp.min(jnp.where(tile == t_max, io_ref[...], float(_K)),
                    axis=1, keepdims=True)
    better = t_max > m_ref[...]
    a_ref[...] = jnp.where(better, t_arg + (k * _TK).astype(jnp.float32),
                           a_ref[...])
    m_ref[...] = jnp.where(better, t_max, m_ref[...])

    @pl.when(k == _KT - 1)
    def _fin():
        idx_ref[0] = a_ref[...].astype(jnp.int32)


def _phase1(xnb, sxn3, cnb, scn3):
    iota_row = jnp.arange(_TK, dtype=jnp.float32).reshape(1, _TK)
    return pl.pallas_call(
        _phase1_body,
        grid=(_NT, _KT),
        in_specs=[
            pl.BlockSpec((_S, _E), lambda t, k: (t, 0)),
            pl.BlockSpec((1, _S, 1), lambda t, k: (t, 0, 0)),
            pl.BlockSpec((_TK, _E), lambda t, k: (k, 0)),
            pl.BlockSpec((1, 1, _TK), lambda t, k: (k, 0, 0)),
            pl.BlockSpec((1, _TK), lambda t, k: (0, 0)),
        ],
        out_specs=[
            pl.BlockSpec((_S, _TK), lambda t, k: (t, k)),
            pl.BlockSpec((1, _S, 1), lambda t, k: (t, 0, 0)),
        ],
        out_shape=[
            jax.ShapeDtypeStruct((_B * _S, _K), jnp.float32),
            jax.ShapeDtypeStruct((_NT, _S, 1), jnp.int32),
        ],
        scratch_shapes=[
            pltpu.VMEM((_S, 1), jnp.float32),
            pltpu.VMEM((_S, 1), jnp.float32),
        ],
    )(xnb, sxn3, cnb, scn3, iota_row)


def _sc_gather(codebook, idx_row):
    mesh = plsc.VectorSubcoreMesh(core_axis_name="core",
                                  subcore_axis_name="subcore")
    n = idx_row.shape[1]
    window = 128

    @pl.kernel(out_type=jax.ShapeDtypeStruct((n, _E), jnp.float32), mesh=mesh)
    def gk(cb_hbm, i_hbm, o_hbm):
        def body(i_vmem, o_vmem):
            pltpu.sync_copy(cb_hbm.at[i_vmem.at[0]], o_vmem)

        pltpu.emit_pipeline(
            body,
            grid=(n // window,),
            in_specs=[pl.BlockSpec((1, window), lambda i: (0, i))],
            out_specs=[pl.BlockSpec((window, _E), lambda i: (i, 0))],
            core_axis_name=("core", "subcore"),
            dimension_semantics=(pltpu.PARALLEL,),
        )(i_hbm, o_hbm)

    return gk(codebook, idx_row)


def _phase3_body(xq_ref, fl_ref, w_ref, bb_ref, out_ref, loss_ref):
    f = fl_ref[...]
    q = xq_ref[...]
    st = f + (q - f)
    o = lax.dot_general(w_ref[...].astype(jnp.bfloat16), st.astype(jnp.bfloat16),
                        (((1,), (1,)), ((), ())),
                        preferred_element_type=jnp.float32) + bb_ref[...]
    out_ref[0] = o
    d = f - q
    l = jnp.sum(d * d) * (1.0 / float(_E * _S))
    loss_ref[...] = jnp.full((1, 1, 128), l, jnp.float32)


def _phase3(xq, flat, w_out, b_out2):
    return pl.pallas_call(
        _phase3_body,
        grid=(_B,),
        in_specs=[
            pl.BlockSpec((_S, _E), lambda b: (b, 0)),
            pl.BlockSpec((_S, _E), lambda b: (b, 0)),
            pl.BlockSpec((_C, _E), lambda b: (0, 0)),
            pl.BlockSpec((_C, 1), lambda b: (0, 0)),
        ],
        out_specs=[
            pl.BlockSpec((1, _C, _S), lambda b: (b, 0, 0)),
            pl.BlockSpec((1, 1, 128), lambda b: (b, 0, 0)),
        ],
        out_shape=[
            jax.ShapeDtypeStruct((_B, _C, _S), jnp.float32),
            jax.ShapeDtypeStruct((_B, 1, 128), jnp.float32),
        ],
    )(xq, flat, w_out, b_out2)


def kernel(x, v_in, g_in, b_in, codebook, v_out, g_out, b_out):
    n_in = jnp.sqrt(jnp.sum(v_in * v_in, axis=1, keepdims=True))
    w_in = g_in * v_in / n_in
    x_proj = jnp.einsum('oc,bcs->bos', w_in, x) + b_in[None, :, None]
    flat = jnp.transpose(x_proj, (0, 2, 1)).reshape(_B * _S, _E)
    xn = flat / jnp.maximum(jnp.linalg.norm(flat, axis=1, keepdims=True), 1e-12)
    cn = codebook / jnp.maximum(jnp.linalg.norm(codebook, axis=1, keepdims=True), 1e-12)
    sxn3 = jnp.sum(xn * xn, axis=1, keepdims=True).reshape(_NT, _S, 1)
    scn3 = jnp.sum(cn * cn, axis=1, keepdims=True).reshape(_KT, 1, _TK)
    n_out = jnp.sqrt(jnp.sum(v_out * v_out, axis=1, keepdims=True))
    w_out = g_out * v_out / n_out
    b_out2 = b_out.reshape(_C, 1)

    logits, idx3 = _phase1(xn.astype(jnp.bfloat16), sxn3,
                           cn.astype(jnp.bfloat16), scn3)
    xq = _sc_gather(codebook, idx3.reshape(1, _B * _S))
    out, loss3 = _phase3(xq, flat, w_out, b_out2)
    x_idxs = idx3.reshape(_B, _S)
    loss = loss3[:, 0, 0]
    return out, logits, x_idxs, loss, loss


# speedup vs baseline: 2.4209x; 1.0004x over previous
import jax
import jax.numpy as jnp
from jax import lax
from jax.experimental import pallas as pl
from jax.experimental.pallas import tpu as pltpu
from jax.experimental.pallas import tpu_sc as plsc

_B, _C, _S = 8, 768, 1024
_E, _K = 256, 8192
_TAU = 1.0
_TK = 1024
_KT = _K // _TK
_NT = (_B * _S) // _S


def _phase1_body(xnb_ref, sx_ref, cnb_ref, sc_ref, io_ref,
                 logits_ref, idx_ref, m_ref, a_ref):
    k = pl.program_id(1)

    @pl.when(k == 0)
    def _init():
        m_ref[...] = jnp.full((_S, 1), -jnp.inf, jnp.float32)
        a_ref[...] = jnp.zeros((_S, 1), jnp.float32)

    dots = lax.dot_general(
        xnb_ref[...], cnb_ref[...], (((1,), (1,)), ((), ())),
        preferred_element_type=jnp.float32)
    tile = (2.0 * dots - sx_ref[0]) - sc_ref[0]
    logits_ref[...] = tile

    t_max = jnp.max(tile, axis=1, keepdims=True)
    t_arg = jnp.min(jnp.where(tile == t_max, io_ref[...], float(_K)),
                    axis=1, keepdims=True)
    better = t_max > m_ref[...]
    a_ref[...] = jnp.where(better, t_arg + (k * _TK).astype(jnp.float32),
                           a_ref[...])
    m_ref[...] = jnp.where(better, t_max, m_ref[...])

    @pl.when(k == _KT - 1)
    def _fin():
        idx_ref[0] = a_ref[...].astype(jnp.int32)


def _phase1(xnb, sxn3, cnb, scn3):
    iota_row = jnp.arange(_TK, dtype=jnp.float32).reshape(1, _TK)
    return pl.pallas_call(
        _phase1_body,
        grid=(_NT, _KT),
        in_specs=[
            pl.BlockSpec((_S, _E), lambda t, k: (t, 0)),
            pl.BlockSpec((1, _S, 1), lambda t, k: (t, 0, 0)),
            pl.BlockSpec((_TK, _E), lambda t, k: (k, 0)),
            pl.BlockSpec((1, 1, _TK), lambda t, k: (k, 0, 0)),
            pl.BlockSpec((1, _TK), lambda t, k: (0, 0)),
        ],
        out_specs=[
            pl.BlockSpec((_S, _TK), lambda t, k: (t, k)),
            pl.BlockSpec((1, _S, 1), lambda t, k: (t, 0, 0)),
        ],
        out_shape=[
            jax.ShapeDtypeStruct((_B * _S, _K), jnp.float32),
            jax.ShapeDtypeStruct((_NT, _S, 1), jnp.int32),
        ],
        scratch_shapes=[
            pltpu.VMEM((_S, 1), jnp.float32),
            pltpu.VMEM((_S, 1), jnp.float32),
        ],
    )(xnb, sxn3, cnb, scn3, iota_row)


def _sc_gather(codebook, idx_row):
    mesh = plsc.VectorSubcoreMesh(core_axis_name="core",
                                  subcore_axis_name="subcore")
    n = idx_row.shape[1]
    window = 128

    @pl.kernel(out_type=jax.ShapeDtypeStruct((n, _E), jnp.float32), mesh=mesh)
    def gk(cb_hbm, i_hbm, o_hbm):
        def body(i_vmem, o_vmem):
            pltpu.sync_copy(cb_hbm.at[i_vmem.at[0]], o_vmem)

        pltpu.emit_pipeline(
            body,
            grid=(n // window,),
            in_specs=[pl.BlockSpec((1, window), lambda i: (0, i))],
            out_specs=[pl.BlockSpec((window, _E), lambda i: (i, 0))],
            core_axis_name=("core", "subcore"),
            dimension_semantics=(pltpu.PARALLEL,),
        )(i_hbm, o_hbm)

    return gk(codebook, idx_row)


def _phase3_body(xq_ref, fl_ref, w_ref, bb_ref, out_ref, loss_ref):
    f = fl_ref[...]
    q = xq_ref[...]
    st = f + (q - f)
    o = lax.dot_general(w_ref[...].astype(jnp.bfloat16), st.astype(jnp.bfloat16),
                        (((1,), (1,)), ((), ())),
                        preferred_element_type=jnp.float32) + bb_ref[...]
    out_ref[0] = o
    d = f - q
    l = jnp.sum(d * d) * (1.0 / float(_E * _S))
    loss_ref[...] = jnp.full((1, 1, 128), l, jnp.float32)


def _phase3(xq, flat, w_out, b_out2):
    return pl.pallas_call(
        _phase3_body,
        grid=(_B,),
        in_specs=[
            pl.BlockSpec((_S, _E), lambda b: (b, 0)),
            pl.BlockSpec((_S, _E), lambda b: (b, 0)),
            pl.BlockSpec((_C, _E), lambda b: (0, 0)),
            pl.BlockSpec((_C, 1), lambda b: (0, 0)),
        ],
        out_specs=[
            pl.BlockSpec((1, _C, _S), lambda b: (b, 0, 0)),
            pl.BlockSpec((1, 1, 128), lambda b: (b, 0, 0)),
        ],
        out_shape=[
            jax.ShapeDtypeStruct((_B, _C, _S), jnp.float32),
            jax.ShapeDtypeStruct((_B, 1, 128), jnp.float32),
        ],
    )(xq, flat, w_out, b_out2)


def kernel(x, v_in, g_in, b_in, codebook, v_out, g_out, b_out):
    n_in = jnp.sqrt(jnp.sum(v_in * v_in, axis=1, keepdims=True))
    w_in = g_in * v_in / n_in
    flat = (jnp.einsum('oc,bcs->bso', w_in, x)
            + b_in[None, None, :]).reshape(_B * _S, _E)
    xn = flat / jnp.maximum(jnp.linalg.norm(flat, axis=1, keepdims=True), 1e-12)
    cn = codebook / jnp.maximum(jnp.linalg.norm(codebook, axis=1, keepdims=True), 1e-12)
    sxn3 = jnp.sum(xn * xn, axis=1, keepdims=True).reshape(_NT, _S, 1)
    scn3 = jnp.sum(cn * cn, axis=1, keepdims=True).reshape(_KT, 1, _TK)
    n_out = jnp.sqrt(jnp.sum(v_out * v_out, axis=1, keepdims=True))
    w_out = g_out * v_out / n_out
    b_out2 = b_out.reshape(_C, 1)

    logits, idx3 = _phase1(xn.astype(jnp.bfloat16), sxn3,
                           cn.astype(jnp.bfloat16), scn3)
    xq = _sc_gather(codebook, idx3.reshape(1, _B * _S))
    out, loss3 = _phase3(xq, flat, w_out, b_out2)
    x_idxs = idx3.reshape(_B, _S)
    loss = loss3[:, 0, 0]
    return out, logits, x_idxs, loss, loss


# speedup vs baseline: 2.5156x; 1.0391x over previous
import jax
import jax.numpy as jnp
from jax import lax
from jax.experimental import pallas as pl
from jax.experimental.pallas import tpu as pltpu
from jax.experimental.pallas import tpu_sc as plsc

_B, _C, _S = 8, 768, 1024
_E, _K = 256, 8192
_TAU = 1.0
_TK = 1024
_KT = _K // _TK
_NT = (_B * _S) // _S


def _phase1_body(xnb_ref, sx_ref, cnb_ref, sc_ref, io_ref,
                 logits_ref, idx_ref, m_ref, a_ref):
    t = pl.program_id(0)
    k = pl.program_id(1)

    @pl.when(k == 0)
    def _init():
        m_ref[...] = jnp.full((_S, 1), -jnp.inf, jnp.float32)
        a_ref[...] = jnp.zeros((_S, 1), jnp.float32)

    dots = lax.dot_general(
        xnb_ref[pl.ds(t * _S, _S), :], cnb_ref[pl.ds(k * _TK, _TK), :],
        (((1,), (1,)), ((), ())),
        preferred_element_type=jnp.float32)
    tile = (2.0 * dots - sx_ref[0]) - sc_ref[0]
    logits_ref[...] = tile

    t_max = jnp.max(tile, axis=1, keepdims=True)
    t_arg = jnp.min(jnp.where(tile == t_max, io_ref[...], float(_K)),
                    axis=1, keepdims=True)
    better = t_max > m_ref[...]
    a_ref[...] = jnp.where(better, t_arg + (k * _TK).astype(jnp.float32),
                           a_ref[...])
    m_ref[...] = jnp.where(better, t_max, m_ref[...])

    @pl.when(k == _KT - 1)
    def _fin():
        idx_ref[0] = a_ref[...].astype(jnp.int32)


def _phase1(xnb, sxn3, cnb, scn3):
    iota_row = jnp.arange(_TK, dtype=jnp.float32).reshape(1, _TK)
    return pl.pallas_call(
        _phase1_body,
        grid=(_NT, _KT),
        in_specs=[
            pl.BlockSpec((_B * _S, _E), lambda t, k: (0, 0)),
            pl.BlockSpec((1, _S, 1), lambda t, k: (t, 0, 0)),
            pl.BlockSpec((_K, _E), lambda t, k: (0, 0)),
            pl.BlockSpec((1, 1, _TK), lambda t, k: (k, 0, 0)),
            pl.BlockSpec((1, _TK), lambda t, k: (0, 0)),
        ],
        out_specs=[
            pl.BlockSpec((_S, _TK), lambda t, k: (t, k)),
            pl.BlockSpec((1, _S, 1), lambda t, k: (t, 0, 0)),
        ],
        out_shape=[
            jax.ShapeDtypeStruct((_B * _S, _K), jnp.float32),
            jax.ShapeDtypeStruct((_NT, _S, 1), jnp.int32),
        ],
        scratch_shapes=[
            pltpu.VMEM((_S, 1), jnp.float32),
            pltpu.VMEM((_S, 1), jnp.float32),
        ],
    )(xnb, sxn3, cnb, scn3, iota_row)


def _sc_gather(codebook, idx_row):
    mesh = plsc.VectorSubcoreMesh(core_axis_name="core",
                                  subcore_axis_name="subcore")
    n = idx_row.shape[1]
    window = 128

    @pl.kernel(out_type=jax.ShapeDtypeStruct((n, _E), jnp.float32), mesh=mesh)
    def gk(cb_hbm, i_hbm, o_hbm):
        def body(i_vmem, o_vmem):
            pltpu.sync_copy(cb_hbm.at[i_vmem.at[0]], o_vmem)

        pltpu.emit_pipeline(
            body,
            grid=(n // window,),
            in_specs=[pl.BlockSpec((1, window), lambda i: (0, i))],
            out_specs=[pl.BlockSpec((window, _E), lambda i: (i, 0))],
            core_axis_name=("core", "subcore"),
            dimension_semantics=(pltpu.PARALLEL,),
        )(i_hbm, o_hbm)

    return gk(codebook, idx_row)


def _phase3_body(xq_ref, fl_ref, w_ref, bb_ref, out_ref, loss_ref):
    f = fl_ref[...]
    q = xq_ref[...]
    st = f + (q - f)
    o = lax.dot_general(w_ref[...].astype(jnp.bfloat16), st.astype(jnp.bfloat16),
                        (((1,), (1,)), ((), ())),
                        preferred_element_type=jnp.float32) + bb_ref[...]
    out_ref[0] = o
    d = f - q
    l = jnp.sum(d * d) * (1.0 / float(_E * _S))
    loss_ref[...] = jnp.full((1, 1, 128), l, jnp.float32)


def _phase3(xq, flat, w_out, b_out2):
    return pl.pallas_call(
        _phase3_body,
        grid=(_B,),
        in_specs=[
            pl.BlockSpec((_S, _E), lambda b: (b, 0)),
            pl.BlockSpec((_S, _E), lambda b: (b, 0)),
            pl.BlockSpec((_C, _E), lambda b: (0, 0)),
            pl.BlockSpec((_C, 1), lambda b: (0, 0)),
        ],
        out_specs=[
            pl.BlockSpec((1, _C, _S), lambda b: (b, 0, 0)),
            pl.BlockSpec((1, 1, 128), lambda b: (b, 0, 0)),
        ],
        out_shape=[
            jax.ShapeDtypeStruct((_B, _C, _S), jnp.float32),
            jax.ShapeDtypeStruct((_B, 1, 128), jnp.float32),
        ],
    )(xq, flat, w_out, b_out2)


def kernel(x, v_in, g_in, b_in, codebook, v_out, g_out, b_out):
    n_in = jnp.sqrt(jnp.sum(v_in * v_in, axis=1, keepdims=True))
    w_in = g_in * v_in / n_in
    flat = (jnp.einsum('oc,bcs->bso', w_in, x)
            + b_in[None, None, :]).reshape(_B * _S, _E)
    xn = flat / jnp.maximum(jnp.linalg.norm(flat, axis=1, keepdims=True), 1e-12)
    cn = codebook / jnp.maximum(jnp.linalg.norm(codebook, axis=1, keepdims=True), 1e-12)
    sxn3 = jnp.sum(xn * xn, axis=1, keepdims=True).reshape(_NT, _S, 1)
    scn3 = jnp.sum(cn * cn, axis=1, keepdims=True).reshape(_KT, 1, _TK)
    n_out = jnp.sqrt(jnp.sum(v_out * v_out, axis=1, keepdims=True))
    w_out = g_out * v_out / n_out
    b_out2 = b_out.reshape(_C, 1)

    logits, idx3 = _phase1(xn.astype(jnp.bfloat16), sxn3,
                           cn.astype(jnp.bfloat16), scn3)
    xq = _sc_gather(codebook, idx3.reshape(1, _B * _S))
    out, loss3 = _phase3(xq, flat, w_out, b_out2)
    x_idxs = idx3.reshape(_B, _S)
    loss = loss3[:, 0, 0]
    return out, logits, x_idxs, loss, loss


# speedup vs baseline: 2.7240x; 1.0828x over previous
import jax
import jax.numpy as jnp
from jax import lax
from jax.experimental import pallas as pl
from jax.experimental.pallas import tpu as pltpu
from jax.experimental.pallas import tpu_sc as plsc

_B, _C, _S = 8, 768, 1024
_E, _K = 256, 8192
_TAU = 1.0
_TK = 2048
_KT = _K // _TK
_NT = (_B * _S) // _S


def _phase1_body(xnb_ref, sx_ref, cnb_ref, sc_ref, io_ref,
                 logits_ref, idx_ref, m_ref, a_ref):
    t = pl.program_id(0)
    k = pl.program_id(1)

    @pl.when(k == 0)
    def _init():
        m_ref[...] = jnp.full((_S, 1), -jnp.inf, jnp.float32)
        a_ref[...] = jnp.zeros((_S, 1), jnp.float32)

    dots = lax.dot_general(
        xnb_ref[pl.ds(t * _S, _S), :], cnb_ref[pl.ds(k * _TK, _TK), :],
        (((1,), (1,)), ((), ())),
        preferred_element_type=jnp.float32)
    tile = (2.0 * dots - sx_ref[0]) - sc_ref[0]
    logits_ref[...] = tile

    t_max = jnp.max(tile, axis=1, keepdims=True)
    t_arg = jnp.min(jnp.where(tile == t_max, io_ref[...], float(_K)),
                    axis=1, keepdims=True)
    better = t_max > m_ref[...]
    a_ref[...] = jnp.where(better, t_arg + (k * _TK).astype(jnp.float32),
                           a_ref[...])
    m_ref[...] = jnp.where(better, t_max, m_ref[...])

    @pl.when(k == _KT - 1)
    def _fin():
        idx_ref[0] = a_ref[...].astype(jnp.int32)


def _phase1(xnb, sxn3, cnb, scn3):
    iota_row = jnp.arange(_TK, dtype=jnp.float32).reshape(1, _TK)
    return pl.pallas_call(
        _phase1_body,
        grid=(_NT, _KT),
        in_specs=[
            pl.BlockSpec((_B * _S, _E), lambda t, k: (0, 0)),
            pl.BlockSpec((1, _S, 1), lambda t, k: (t, 0, 0)),
            pl.BlockSpec((_K, _E), lambda t, k: (0, 0)),
            pl.BlockSpec((1, 1, _TK), lambda t, k: (k, 0, 0)),
            pl.BlockSpec((1, _TK), lambda t, k: (0, 0)),
        ],
        out_specs=[
            pl.BlockSpec((_S, _TK), lambda t, k: (t, k)),
            pl.BlockSpec((1, _S, 1), lambda t, k: (t, 0, 0)),
        ],
        out_shape=[
            jax.ShapeDtypeStruct((_B * _S, _K), jnp.float32),
            jax.ShapeDtypeStruct((_NT, _S, 1), jnp.int32),
        ],
        scratch_shapes=[
            pltpu.VMEM((_S, 1), jnp.float32),
            pltpu.VMEM((_S, 1), jnp.float32),
        ],
        compiler_params=pltpu.CompilerParams(
            dimension_semantics=("parallel", "arbitrary")),
    )(xnb, sxn3, cnb, scn3, iota_row)


def _sc_gather(codebook, idx_row):
    mesh = plsc.VectorSubcoreMesh(core_axis_name="core",
                                  subcore_axis_name="subcore")
    n = idx_row.shape[1]
    window = 128

    @pl.kernel(out_type=jax.ShapeDtypeStruct((n, _E), jnp.float32), mesh=mesh)
    def gk(cb_hbm, i_hbm, o_hbm):
        def body(i_vmem, o_vmem):
            pltpu.sync_copy(cb_hbm.at[i_vmem.at[0]], o_vmem)

        pltpu.emit_pipeline(
            body,
            grid=(n // window,),
            in_specs=[pl.BlockSpec((1, window), lambda i: (0, i))],
            out_specs=[pl.BlockSpec((window, _E), lambda i: (i, 0))],
            core_axis_name=("core", "subcore"),
            dimension_semantics=(pltpu.PARALLEL,),
        )(i_hbm, o_hbm)

    return gk(codebook, idx_row)


def _phase3_body(xq_ref, fl_ref, w_ref, bb_ref, out_ref, loss_ref):
    f = fl_ref[...]
    q = xq_ref[...]
    st = f + (q - f)
    o = lax.dot_general(w_ref[...].astype(jnp.bfloat16), st.astype(jnp.bfloat16),
                        (((1,), (1,)), ((), ())),
                        preferred_element_type=jnp.float32) + bb_ref[...]
    out_ref[0] = o
    d = f - q
    l = jnp.sum(d * d) * (1.0 / float(_E * _S))
    loss_ref[...] = jnp.full((1, 1, 128), l, jnp.float32)


def _phase3(xq, flat, w_out, b_out2):
    return pl.pallas_call(
        _phase3_body,
        grid=(_B,),
        in_specs=[
            pl.BlockSpec((_S, _E), lambda b: (b, 0)),
            pl.BlockSpec((_S, _E), lambda b: (b, 0)),
            pl.BlockSpec((_C, _E), lambda b: (0, 0)),
            pl.BlockSpec((_C, 1), lambda b: (0, 0)),
        ],
        out_specs=[
            pl.BlockSpec((1, _C, _S), lambda b: (b, 0, 0)),
            pl.BlockSpec((1, 1, 128), lambda b: (b, 0, 0)),
        ],
        out_shape=[
            jax.ShapeDtypeStruct((_B, _C, _S), jnp.float32),
            jax.ShapeDtypeStruct((_B, 1, 128), jnp.float32),
        ],
    )(xq, flat, w_out, b_out2)


def kernel(x, v_in, g_in, b_in, codebook, v_out, g_out, b_out):
    n_in = jnp.sqrt(jnp.sum(v_in * v_in, axis=1, keepdims=True))
    w_in = g_in * v_in / n_in
    flat = (jnp.einsum('oc,bcs->bso', w_in, x)
            + b_in[None, None, :]).reshape(_B * _S, _E)
    xn = flat / jnp.maximum(jnp.linalg.norm(flat, axis=1, keepdims=True), 1e-12)
    cn = codebook / jnp.maximum(jnp.linalg.norm(codebook, axis=1, keepdims=True), 1e-12)
    sxn3 = jnp.sum(xn * xn, axis=1, keepdims=True).reshape(_NT, _S, 1)
    scn3 = jnp.sum(cn * cn, axis=1, keepdims=True).reshape(_KT, 1, _TK)
    n_out = jnp.sqrt(jnp.sum(v_out * v_out, axis=1, keepdims=True))
    w_out = g_out * v_out / n_out
    b_out2 = b_out.reshape(_C, 1)

    logits, idx3 = _phase1(xn.astype(jnp.bfloat16), sxn3,
                           cn.astype(jnp.bfloat16), scn3)
    xq = _sc_gather(codebook, idx3.reshape(1, _B * _S))
    out, loss3 = _phase3(xq, flat, w_out, b_out2)
    x_idxs = idx3.reshape(_B, _S)
    loss = loss3[:, 0, 0]
    return out, logits, x_idxs, loss, loss


# speedup vs baseline: 2.8076x; 1.0307x over previous
import jax
import jax.numpy as jnp
from jax import lax
from jax.experimental import pallas as pl
from jax.experimental.pallas import tpu as pltpu
from jax.experimental.pallas import tpu_sc as plsc

_B, _C, _S = 8, 768, 1024
_E, _K = 256, 8192
_TAU = 1.0
_TK = 4096
_KT = _K // _TK
_NT = (_B * _S) // _S


def _phase1_body(xnb_ref, sx_ref, cnb_ref, sc_ref, io_ref,
                 logits_ref, idx_ref, m_ref, a_ref):
    t = pl.program_id(0)
    k = pl.program_id(1)

    @pl.when(k == 0)
    def _init():
        m_ref[...] = jnp.full((_S, 1), -jnp.inf, jnp.float32)
        a_ref[...] = jnp.zeros((_S, 1), jnp.float32)

    dots = lax.dot_general(
        xnb_ref[pl.ds(t * _S, _S), :], cnb_ref[pl.ds(k * _TK, _TK), :],
        (((1,), (1,)), ((), ())),
        preferred_element_type=jnp.float32)
    tile = (2.0 * dots - sx_ref[0]) - sc_ref[0]
    logits_ref[...] = tile

    t_max = jnp.max(tile, axis=1, keepdims=True)
    t_arg = jnp.min(jnp.where(tile == t_max, io_ref[...], float(_K)),
                    axis=1, keepdims=True)
    better = t_max > m_ref[...]
    a_ref[...] = jnp.where(better, t_arg + (k * _TK).astype(jnp.float32),
                           a_ref[...])
    m_ref[...] = jnp.where(better, t_max, m_ref[...])

    @pl.when(k == _KT - 1)
    def _fin():
        idx_ref[0] = a_ref[...].astype(jnp.int32)


def _phase1(xnb, sxn3, cnb, scn3):
    iota_row = jnp.arange(_TK, dtype=jnp.float32).reshape(1, _TK)
    return pl.pallas_call(
        _phase1_body,
        grid=(_NT, _KT),
        in_specs=[
            pl.BlockSpec((_B * _S, _E), lambda t, k: (0, 0)),
            pl.BlockSpec((1, _S, 1), lambda t, k: (t, 0, 0)),
            pl.BlockSpec((_K, _E), lambda t, k: (0, 0)),
            pl.BlockSpec((1, 1, _TK), lambda t, k: (k, 0, 0)),
            pl.BlockSpec((1, _TK), lambda t, k: (0, 0)),
        ],
        out_specs=[
            pl.BlockSpec((_S, _TK), lambda t, k: (t, k)),
            pl.BlockSpec((1, _S, 1), lambda t, k: (t, 0, 0)),
        ],
        out_shape=[
            jax.ShapeDtypeStruct((_B * _S, _K), jnp.float32),
            jax.ShapeDtypeStruct((_NT, _S, 1), jnp.int32),
        ],
        scratch_shapes=[
            pltpu.VMEM((_S, 1), jnp.float32),
            pltpu.VMEM((_S, 1), jnp.float32),
        ],
        compiler_params=pltpu.CompilerParams(
            dimension_semantics=("parallel", "arbitrary")),
    )(xnb, sxn3, cnb, scn3, iota_row)


def _sc_gather(codebook, idx_row):
    mesh = plsc.VectorSubcoreMesh(core_axis_name="core",
                                  subcore_axis_name="subcore")
    n = idx_row.shape[1]
    window = 128

    @pl.kernel(out_type=jax.ShapeDtypeStruct((n, _E), jnp.float32), mesh=mesh)
    def gk(cb_hbm, i_hbm, o_hbm):
        def body(i_vmem, o_vmem):
            pltpu.sync_copy(cb_hbm.at[i_vmem.at[0]], o_vmem)

        pltpu.emit_pipeline(
            body,
            grid=(n // window,),
            in_specs=[pl.BlockSpec((1, window), lambda i: (0, i))],
            out_specs=[pl.BlockSpec((window, _E), lambda i: (i, 0))],
            core_axis_name=("core", "subcore"),
            dimension_semantics=(pltpu.PARALLEL,),
        )(i_hbm, o_hbm)

    return gk(codebook, idx_row)


def _phase3_body(xq_ref, fl_ref, w_ref, bb_ref, out_ref, loss_ref):
    f = fl_ref[...]
    q = xq_ref[...]
    st = f + (q - f)
    o = lax.dot_general(w_ref[...].astype(jnp.bfloat16), st.astype(jnp.bfloat16),
                        (((1,), (1,)), ((), ())),
                        preferred_element_type=jnp.float32) + bb_ref[...]
    out_ref[0] = o
    d = f - q
    l = jnp.sum(d * d) * (1.0 / float(_E * _S))
    loss_ref[...] = jnp.full((1, 1, 128), l, jnp.float32)


def _phase3(xq, flat, w_out, b_out2):
    return pl.pallas_call(
        _phase3_body,
        grid=(_B,),
        in_specs=[
            pl.BlockSpec((_S, _E), lambda b: (b, 0)),
            pl.BlockSpec((_S, _E), lambda b: (b, 0)),
            pl.BlockSpec((_C, _E), lambda b: (0, 0)),
            pl.BlockSpec((_C, 1), lambda b: (0, 0)),
        ],
        out_specs=[
            pl.BlockSpec((1, _C, _S), lambda b: (b, 0, 0)),
            pl.BlockSpec((1, 1, 128), lambda b: (b, 0, 0)),
        ],
        out_shape=[
            jax.ShapeDtypeStruct((_B, _C, _S), jnp.float32),
            jax.ShapeDtypeStruct((_B, 1, 128), jnp.float32),
        ],
    )(xq, flat, w_out, b_out2)


def kernel(x, v_in, g_in, b_in, codebook, v_out, g_out, b_out):
    n_in = jnp.sqrt(jnp.sum(v_in * v_in, axis=1, keepdims=True))
    w_in = g_in * v_in / n_in
    flat = (jnp.einsum('oc,bcs->bso', w_in, x)
            + b_in[None, None, :]).reshape(_B * _S, _E)
    xn = flat / jnp.maximum(jnp.linalg.norm(flat, axis=1, keepdims=True), 1e-12)
    cn = codebook / jnp.maximum(jnp.linalg.norm(codebook, axis=1, keepdims=True), 1e-12)
    sxn3 = jnp.sum(xn * xn, axis=1, keepdims=True).reshape(_NT, _S, 1)
    scn3 = jnp.sum(cn * cn, axis=1, keepdims=True).reshape(_KT, 1, _TK)
    n_out = jnp.sqrt(jnp.sum(v_out * v_out, axis=1, keepdims=True))
    w_out = g_out * v_out / n_out
    b_out2 = b_out.reshape(_C, 1)

    logits, idx3 = _phase1(xn.astype(jnp.bfloat16), sxn3,
                           cn.astype(jnp.bfloat16), scn3)
    xq = _sc_gather(codebook, idx3.reshape(1, _B * _S))
    out, loss3 = _phase3(xq, flat, w_out, b_out2)
    x_idxs = idx3.reshape(_B, _S)
    loss = loss3[:, 0, 0]
    return out, logits, x_idxs, loss, loss


# speedup vs baseline: 2.8697x; 1.0221x over previous
import jax
import jax.numpy as jnp
from jax import lax
from jax.experimental import pallas as pl
from jax.experimental.pallas import tpu as pltpu
from jax.experimental.pallas import tpu_sc as plsc

_B, _C, _S = 8, 768, 1024
_E, _K = 256, 8192
_TAU = 1.0
_TK = 8192
_KT = _K // _TK
_TS = 512
_NT = (_B * _S) // _TS


def _phase1_body(xnb_ref, sx_ref, cnb_ref, sc_ref, io_ref,
                 logits_ref, idx_ref):
    t = pl.program_id(0)

    dots = lax.dot_general(
        xnb_ref[pl.ds(t * _TS, _TS), :], cnb_ref[...],
        (((1,), (1,)), ((), ())),
        preferred_element_type=jnp.float32)
    tile = (2.0 * dots - sx_ref[0]) - sc_ref[0]
    logits_ref[...] = tile

    t_max = jnp.max(tile, axis=1, keepdims=True)
    t_arg = jnp.min(jnp.where(tile == t_max, io_ref[...], float(_K)),
                    axis=1, keepdims=True)
    idx_ref[0] = t_arg.astype(jnp.int32)


def _phase1(xnb, sxn3, cnb, scn3):
    iota_row = jnp.arange(_TK, dtype=jnp.float32).reshape(1, _TK)
    return pl.pallas_call(
        _phase1_body,
        grid=(_NT,),
        in_specs=[
            pl.BlockSpec((_B * _S, _E), lambda t: (0, 0)),
            pl.BlockSpec((1, _TS, 1), lambda t: (t, 0, 0)),
            pl.BlockSpec((_K, _E), lambda t: (0, 0)),
            pl.BlockSpec((1, 1, _TK), lambda t: (0, 0, 0)),
            pl.BlockSpec((1, _TK), lambda t: (0, 0)),
        ],
        out_specs=[
            pl.BlockSpec((_TS, _TK), lambda t: (t, 0)),
            pl.BlockSpec((1, _TS, 1), lambda t: (t, 0, 0)),
        ],
        out_shape=[
            jax.ShapeDtypeStruct((_B * _S, _K), jnp.float32),
            jax.ShapeDtypeStruct((_NT, _TS, 1), jnp.int32),
        ],
        compiler_params=pltpu.CompilerParams(
            dimension_semantics=("parallel",)),
    )(xnb, sxn3, cnb, scn3, iota_row)


def _sc_gather(codebook, idx_row):
    mesh = plsc.VectorSubcoreMesh(core_axis_name="core",
                                  subcore_axis_name="subcore")
    n = idx_row.shape[1]
    window = 128

    @pl.kernel(out_type=jax.ShapeDtypeStruct((n, _E), jnp.float32), mesh=mesh)
    def gk(cb_hbm, i_hbm, o_hbm):
        def body(i_vmem, o_vmem):
            pltpu.sync_copy(cb_hbm.at[i_vmem.at[0]], o_vmem)

        pltpu.emit_pipeline(
            body,
            grid=(n // window,),
            in_specs=[pl.BlockSpec((1, window), lambda i: (0, i))],
            out_specs=[pl.BlockSpec((window, _E), lambda i: (i, 0))],
            core_axis_name=("core", "subcore"),
            dimension_semantics=(pltpu.PARALLEL,),
        )(i_hbm, o_hbm)

    return gk(codebook, idx_row)


def _phase3_body(xq_ref, fl_ref, w_ref, bb_ref, out_ref, loss_ref):
    f = fl_ref[...]
    q = xq_ref[...]
    st = f + (q - f)
    o = lax.dot_general(w_ref[...].astype(jnp.bfloat16), st.astype(jnp.bfloat16),
                        (((1,), (1,)), ((), ())),
                        preferred_element_type=jnp.float32) + bb_ref[...]
    out_ref[0] = o
    d = f - q
    l = jnp.sum(d * d) * (1.0 / float(_E * _S))
    loss_ref[...] = jnp.full((1, 1, 128), l, jnp.float32)


def _phase3(xq, flat, w_out, b_out2):
    return pl.pallas_call(
        _phase3_body,
        grid=(_B,),
        in_specs=[
            pl.BlockSpec((_S, _E), lambda b: (b, 0)),
            pl.BlockSpec((_S, _E), lambda b: (b, 0)),
            pl.BlockSpec((_C, _E), lambda b: (0, 0)),
            pl.BlockSpec((_C, 1), lambda b: (0, 0)),
        ],
        out_specs=[
            pl.BlockSpec((1, _C, _S), lambda b: (b, 0, 0)),
            pl.BlockSpec((1, 1, 128), lambda b: (b, 0, 0)),
        ],
        out_shape=[
            jax.ShapeDtypeStruct((_B, _C, _S), jnp.float32),
            jax.ShapeDtypeStruct((_B, 1, 128), jnp.float32),
        ],
    )(xq, flat, w_out, b_out2)


def kernel(x, v_in, g_in, b_in, codebook, v_out, g_out, b_out):
    n_in = jnp.sqrt(jnp.sum(v_in * v_in, axis=1, keepdims=True))
    w_in = g_in * v_in / n_in
    flat = (jnp.einsum('oc,bcs->bso', w_in, x)
            + b_in[None, None, :]).reshape(_B * _S, _E)
    xn = flat / jnp.maximum(jnp.linalg.norm(flat, axis=1, keepdims=True), 1e-12)
    cn = codebook / jnp.maximum(jnp.linalg.norm(codebook, axis=1, keepdims=True), 1e-12)
    sxn3 = jnp.sum(xn * xn, axis=1, keepdims=True).reshape(_NT, _TS, 1)
    scn3 = jnp.sum(cn * cn, axis=1, keepdims=True).reshape(_KT, 1, _TK)
    n_out = jnp.sqrt(jnp.sum(v_out * v_out, axis=1, keepdims=True))
    w_out = g_out * v_out / n_out
    b_out2 = b_out.reshape(_C, 1)

    logits, idx3 = _phase1(xn.astype(jnp.bfloat16), sxn3,
                           cn.astype(jnp.bfloat16), scn3)
    xq = _sc_gather(codebook, idx3.reshape(1, _B * _S))
    out, loss3 = _phase3(xq, flat, w_out, b_out2)
    x_idxs = idx3.reshape(_B, _S)
    loss = loss3[:, 0, 0]
    return out, logits, x_idxs, loss, loss
